# baseline (speedup 1.0000x reference)
import functools

import jax
import jax.numpy as jnp
from jax import lax
from jax.experimental import pallas as pl
from jax.experimental.pallas import tpu as pltpu
from jax.experimental.pallas import tpu_sc as plsc

_NC = 2
_NS = 16
_NW = _NC * _NS
_W = 128
_K = 8192


def _pad_block(a_ref, o_ref):
    t = jnp.swapaxes(a_ref[...], 0, 1)
    o_ref[...] = jnp.concatenate([t, t], axis=1)


def kernel(word_input, embed_weight):
    B, S = word_input.shape
    V, D = embed_weight.shape
    N = B * S
    idx = word_input.T.reshape(N)
    table_t = embed_weight.T

    table_p = pl.pallas_call(
        _pad_block,
        grid=(pl.cdiv(V, _K),),
        in_specs=[pl.BlockSpec((D, _K), lambda i: (0, i))],
        out_specs=pl.BlockSpec((_K, 2 * D), lambda i: (i, 0)),
        out_shape=jax.ShapeDtypeStruct((V, 2 * D), embed_weight.dtype),
        compiler_params=pltpu.CompilerParams(
            dimension_semantics=("parallel",)),
    )(table_t)

    n_per_w = N // _NW
    chunks = n_per_w // _W
    mesh = plsc.VectorSubcoreMesh(core_axis_name="c", subcore_axis_name="s")

    @functools.partial(
        pl.kernel,
        out_type=jax.ShapeDtypeStruct((N, 2 * D), embed_weight.dtype),
        mesh=mesh,
        scratch_types=[
            pltpu.VMEM((n_per_w,), jnp.int32),
            pltpu.VMEM((2, _W, 2 * D), jnp.float32),
            pltpu.SemaphoreType.DMA,
            pltpu.SemaphoreType.DMA,
        ],
    )
    def gather_kernel(table_hbm, idx_hbm, out_hbm, idx_v, rows_v, sem0, sem1):
        wid = lax.axis_index("s") * _NC + lax.axis_index("c")
        base = wid * n_per_w
        pltpu.sync_copy(idx_hbm.at[pl.ds(base, n_per_w)], idx_v)

        def fire(c, buf, sem):
            win = idx_v.at[pl.ds(c * _W, _W)]
            pltpu.async_copy(table_hbm.at[win], rows_v.at[buf], sem)

        def drain(c, buf, sem):
            win = idx_v.at[pl.ds(c * _W, _W)]
            pltpu.make_async_copy(table_hbm.at[win], rows_v.at[buf], sem).wait()
            pltpu.sync_copy(rows_v.at[buf], out_hbm.at[pl.ds(base + c * _W, _W)])

        fire(0, 0, sem0)

        @pl.loop(0, chunks // 2 - 1)
        def _(h):
            c = 2 * h
            fire(c + 1, 1, sem1)
            drain(c, 0, sem0)
            fire(c + 2, 0, sem0)
            drain(c + 1, 1, sem1)

        fire(chunks - 1, 1, sem1)
        drain(chunks - 2, 0, sem0)
        drain(chunks - 1, 1, sem1)

    fused = gather_kernel(table_p, idx)
    out = fused.reshape(S, B, 2 * D)[:, :, :D]
    return jnp.transpose(out, (1, 0, 2))


# speedup vs baseline: 1.1226x; 1.1226x over previous
import functools

import jax
import jax.numpy as jnp
from jax import lax
from jax.experimental import pallas as pl
from jax.experimental.pallas import tpu as pltpu
from jax.experimental.pallas import tpu_sc as plsc

_NC = 2
_NS = 16
_NW = _NC * _NS
_W = 128
_K = 8192


def _pad_block(a_ref, o_ref):
    a = a_ref[...]
    eye = jnp.eye(a.shape[0], dtype=a.dtype)
    eye2 = jnp.concatenate([eye, eye], axis=1)
    o_ref[...] = lax.dot_general(a, eye2, (((0,), (0,)), ((), ())),
                                 preferred_element_type=jnp.float32)


def kernel(word_input, embed_weight):
    B, S = word_input.shape
    V, D = embed_weight.shape
    N = B * S
    idx = word_input.T.reshape(N)
    table_t = embed_weight.T

    table_p = pl.pallas_call(
        _pad_block,
        grid=(pl.cdiv(V, _K),),
        in_specs=[pl.BlockSpec((D, _K), lambda i: (0, i))],
        out_specs=pl.BlockSpec((_K, 2 * D), lambda i: (i, 0)),
        out_shape=jax.ShapeDtypeStruct((V, 2 * D), embed_weight.dtype),
        compiler_params=pltpu.CompilerParams(
            dimension_semantics=("parallel",)),
    )(table_t)

    n_per_w = N // _NW
    chunks = n_per_w // _W
    mesh = plsc.VectorSubcoreMesh(core_axis_name="c", subcore_axis_name="s")

    @functools.partial(
        pl.kernel,
        out_type=jax.ShapeDtypeStruct((N, 2 * D), embed_weight.dtype),
        mesh=mesh,
        scratch_types=[
            pltpu.VMEM((n_per_w,), jnp.int32),
            pltpu.VMEM((2, _W, 2 * D), jnp.float32),
            pltpu.SemaphoreType.DMA,
            pltpu.SemaphoreType.DMA,
        ],
    )
    def gather_kernel(table_hbm, idx_hbm, out_hbm, idx_v, rows_v, sem0, sem1):
        wid = lax.axis_index("s") * _NC + lax.axis_index("c")
        base = wid * n_per_w
        pltpu.sync_copy(idx_hbm.at[pl.ds(base, n_per_w)], idx_v)

        def fire(c, buf, sem):
            win = idx_v.at[pl.ds(c * _W, _W)]
            pltpu.async_copy(table_hbm.at[win], rows_v.at[buf], sem)

        def drain(c, buf, sem):
            win = idx_v.at[pl.ds(c * _W, _W)]
            pltpu.make_async_copy(table_hbm.at[win], rows_v.at[buf], sem).wait()
            pltpu.sync_copy(rows_v.at[buf], out_hbm.at[pl.ds(base + c * _W, _W)])

        fire(0, 0, sem0)

        @pl.loop(0, chunks // 2 - 1)
        def _(h):
            c = 2 * h
            fire(c + 1, 1, sem1)
            drain(c, 0, sem0)
            fire(c + 2, 0, sem0)
            drain(c + 1, 1, sem1)

        fire(chunks - 1, 1, sem1)
        drain(chunks - 2, 0, sem0)
        drain(chunks - 1, 1, sem1)

    fused = gather_kernel(table_p, idx)
    out = fused.reshape(S, B, 2 * D)[:, :, :D]
    return jnp.transpose(out, (1, 0, 2))


# speedup vs baseline: 1.1424x; 1.0176x over previous
import functools

import jax
import jax.numpy as jnp
from jax import lax
from jax.experimental import pallas as pl
from jax.experimental.pallas import tpu as pltpu
from jax.experimental.pallas import tpu_sc as plsc

_NC = 2
_NS = 16
_NW = _NC * _NS
_W = 128
_K = 8192


def _pad_block(a_ref, o_ref):
    o_ref[:, : a_ref.shape[0]] = jnp.swapaxes(a_ref[...], 0, 1)


def kernel(word_input, embed_weight):
    B, S = word_input.shape
    V, D = embed_weight.shape
    N = B * S
    idx = word_input.T.reshape(N)
    table_t = embed_weight.T

    table_p = pl.pallas_call(
        _pad_block,
        grid=(pl.cdiv(V, _K),),
        in_specs=[pl.BlockSpec((D, _K), lambda i: (0, i))],
        out_specs=pl.BlockSpec((_K, 2 * D), lambda i: (i, 0)),
        out_shape=jax.ShapeDtypeStruct((V, 2 * D), embed_weight.dtype),
        compiler_params=pltpu.CompilerParams(
            dimension_semantics=("parallel",)),
    )(table_t)

    n_per_w = N // _NW
    chunks = n_per_w // _W
    mesh = plsc.VectorSubcoreMesh(core_axis_name="c", subcore_axis_name="s")

    @functools.partial(
        pl.kernel,
        out_type=jax.ShapeDtypeStruct((N, 2 * D), embed_weight.dtype),
        mesh=mesh,
        scratch_types=[
            pltpu.VMEM((n_per_w,), jnp.int32),
            pltpu.VMEM((2, _W, 2 * D), jnp.float32),
            pltpu.SemaphoreType.DMA,
            pltpu.SemaphoreType.DMA,
        ],
    )
    def gather_kernel(table_hbm, idx_hbm, out_hbm, idx_v, rows_v, sem0, sem1):
        wid = lax.axis_index("s") * _NC + lax.axis_index("c")
        base = wid * n_per_w
        pltpu.sync_copy(idx_hbm.at[pl.ds(base, n_per_w)], idx_v)

        def fire(c, buf, sem):
            win = idx_v.at[pl.ds(c * _W, _W)]
            pltpu.async_copy(table_hbm.at[win], rows_v.at[buf], sem)

        def drain(c, buf, sem):
            win = idx_v.at[pl.ds(c * _W, _W)]
            pltpu.make_async_copy(table_hbm.at[win], rows_v.at[buf], sem).wait()
            pltpu.sync_copy(rows_v.at[buf], out_hbm.at[pl.ds(base + c * _W, _W)])

        fire(0, 0, sem0)

        @pl.loop(0, chunks // 2 - 1)
        def _(h):
            c = 2 * h
            fire(c + 1, 1, sem1)
            drain(c, 0, sem0)
            fire(c + 2, 0, sem0)
            drain(c + 1, 1, sem1)

        fire(chunks - 1, 1, sem1)
        drain(chunks - 2, 0, sem0)
        drain(chunks - 1, 1, sem1)

    fused = gather_kernel(table_p, idx)
    out = fused.reshape(S, B, 2 * D)[:, :, :D]
    return jnp.transpose(out, (1, 0, 2))


# speedup vs baseline: 1.1984x; 1.0490x over previous
import functools

import jax
import jax.numpy as jnp
from jax import lax
from jax.experimental import pallas as pl
from jax.experimental.pallas import tpu as pltpu
from jax.experimental.pallas import tpu_sc as plsc

_NC = 2
_NS = 16
_NW = _NC * _NS
_W = 128
_K = 16384


def _pad_block(a_ref, o_ref):
    o_ref[:, : a_ref.shape[0]] = jnp.swapaxes(a_ref[...], 0, 1)


def kernel(word_input, embed_weight):
    B, S = word_input.shape
    V, D = embed_weight.shape
    N = B * S
    idx = word_input.T.reshape(N)
    table_t = embed_weight.T

    table_p = pl.pallas_call(
        _pad_block,
        grid=(pl.cdiv(V, _K),),
        in_specs=[pl.BlockSpec((D, _K), lambda i: (0, i))],
        out_specs=pl.BlockSpec((_K, 2 * D), lambda i: (i, 0)),
        out_shape=jax.ShapeDtypeStruct((V, 2 * D), embed_weight.dtype),
        compiler_params=pltpu.CompilerParams(
            dimension_semantics=("parallel",)),
    )(table_t)

    n_per_w = N // _NW
    chunks = n_per_w // _W
    mesh = plsc.VectorSubcoreMesh(core_axis_name="c", subcore_axis_name="s")

    @functools.partial(
        pl.kernel,
        out_type=jax.ShapeDtypeStruct((N, 2 * D), embed_weight.dtype),
        mesh=mesh,
        scratch_types=[
            pltpu.VMEM((n_per_w,), jnp.int32),
            pltpu.VMEM((2, _W, 2 * D), jnp.float32),
            pltpu.SemaphoreType.DMA,
            pltpu.SemaphoreType.DMA,
        ],
    )
    def gather_kernel(table_hbm, idx_hbm, out_hbm, idx_v, rows_v, sem0, sem1):
        wid = lax.axis_index("s") * _NC + lax.axis_index("c")
        base = wid * n_per_w
        pltpu.sync_copy(idx_hbm.at[pl.ds(base, n_per_w)], idx_v)

        def fire(c, buf, sem):
            win = idx_v.at[pl.ds(c * _W, _W)]
            pltpu.async_copy(table_hbm.at[win], rows_v.at[buf], sem)

        def drain(c, buf, sem):
            win = idx_v.at[pl.ds(c * _W, _W)]
            pltpu.make_async_copy(table_hbm.at[win], rows_v.at[buf], sem).wait()
            pltpu.sync_copy(rows_v.at[buf], out_hbm.at[pl.ds(base + c * _W, _W)])

        fire(0, 0, sem0)

        @pl.loop(0, chunks // 2 - 1)
        def _(h):
            c = 2 * h
            fire(c + 1, 1, sem1)
            drain(c, 0, sem0)
            fire(c + 2, 0, sem0)
            drain(c + 1, 1, sem1)

        fire(chunks - 1, 1, sem1)
        drain(chunks - 2, 0, sem0)
        drain(chunks - 1, 1, sem1)

    fused = gather_kernel(table_p, idx)
    out = fused.reshape(S, B, 2 * D)[:, :, :D]
    return jnp.transpose(out, (1, 0, 2))


# speedup vs baseline: 1.2121x; 1.0115x over previous
import functools

import jax
import jax.numpy as jnp
from jax import lax
from jax.experimental import pallas as pl
from jax.experimental.pallas import tpu as pltpu
from jax.experimental.pallas import tpu_sc as plsc

_NC = 2
_NS = 16
_NW = _NC * _NS
_W = 128
_K = 24576


def _pad_block(a_ref, o_ref):
    o_ref[:, : a_ref.shape[0]] = jnp.swapaxes(a_ref[...], 0, 1)


def kernel(word_input, embed_weight):
    B, S = word_input.shape
    V, D = embed_weight.shape
    N = B * S
    idx = word_input.T.reshape(N)
    table_t = embed_weight.T

    table_p = pl.pallas_call(
        _pad_block,
        grid=(pl.cdiv(V, _K),),
        in_specs=[pl.BlockSpec((D, _K), lambda i: (0, i))],
        out_specs=pl.BlockSpec((_K, 2 * D), lambda i: (i, 0)),
        out_shape=jax.ShapeDtypeStruct((V, 2 * D), embed_weight.dtype),
        compiler_params=pltpu.CompilerParams(
            dimension_semantics=("parallel",)),
    )(table_t)

    n_per_w = N // _NW
    chunks = n_per_w // _W
    mesh = plsc.VectorSubcoreMesh(core_axis_name="c", subcore_axis_name="s")

    @functools.partial(
        pl.kernel,
        out_type=jax.ShapeDtypeStruct((N, 2 * D), embed_weight.dtype),
        mesh=mesh,
        scratch_types=[
            pltpu.VMEM((n_per_w,), jnp.int32),
            pltpu.VMEM((2, _W, 2 * D), jnp.float32),
            pltpu.SemaphoreType.DMA,
            pltpu.SemaphoreType.DMA,
        ],
    )
    def gather_kernel(table_hbm, idx_hbm, out_hbm, idx_v, rows_v, sem0, sem1):
        wid = lax.axis_index("s") * _NC + lax.axis_index("c")
        base = wid * n_per_w
        pltpu.sync_copy(idx_hbm.at[pl.ds(base, n_per_w)], idx_v)

        def fire(c, buf, sem):
            win = idx_v.at[pl.ds(c * _W, _W)]
            pltpu.async_copy(table_hbm.at[win], rows_v.at[buf], sem)

        def drain(c, buf, sem):
            win = idx_v.at[pl.ds(c * _W, _W)]
            pltpu.make_async_copy(table_hbm.at[win], rows_v.at[buf], sem).wait()
            pltpu.sync_copy(rows_v.at[buf], out_hbm.at[pl.ds(base + c * _W, _W)])

        fire(0, 0, sem0)

        @pl.loop(0, chunks // 2 - 1)
        def _(h):
            c = 2 * h
            fire(c + 1, 1, sem1)
            drain(c, 0, sem0)
            fire(c + 2, 0, sem0)
            drain(c + 1, 1, sem1)

        fire(chunks - 1, 1, sem1)
        drain(chunks - 2, 0, sem0)
        drain(chunks - 1, 1, sem1)

    fused = gather_kernel(table_p, idx)
    out = fused.reshape(S, B, 2 * D)[:, :, :D]
    return jnp.transpose(out, (1, 0, 2))


# speedup vs baseline: 1.2171x; 1.0041x over previous
import functools

import jax
import jax.numpy as jnp
from jax import lax
from jax.experimental import pallas as pl
from jax.experimental.pallas import tpu as pltpu
from jax.experimental.pallas import tpu_sc as plsc

_NC = 2
_NS = 16
_NW = _NC * _NS
_W = 128
_K = 32768


def _pad_block(a_ref, o_ref):
    o_ref[:, : a_ref.shape[0]] = jnp.swapaxes(a_ref[...], 0, 1)


def kernel(word_input, embed_weight):
    B, S = word_input.shape
    V, D = embed_weight.shape
    N = B * S
    idx = word_input.T.reshape(N)
    table_t = embed_weight.T

    table_p = pl.pallas_call(
        _pad_block,
        grid=(pl.cdiv(V, _K),),
        in_specs=[pl.BlockSpec((D, _K), lambda i: (0, i))],
        out_specs=pl.BlockSpec((_K, 2 * D), lambda i: (i, 0)),
        out_shape=jax.ShapeDtypeStruct((V, 2 * D), embed_weight.dtype),
        compiler_params=pltpu.CompilerParams(
            dimension_semantics=("parallel",),
            vmem_limit_bytes=56 * 1024 * 1024),
    )(table_t)

    n_per_w = N // _NW
    chunks = n_per_w // _W
    mesh = plsc.VectorSubcoreMesh(core_axis_name="c", subcore_axis_name="s")

    @functools.partial(
        pl.kernel,
        out_type=jax.ShapeDtypeStruct((N, 2 * D), embed_weight.dtype),
        mesh=mesh,
        scratch_types=[
            pltpu.VMEM((n_per_w,), jnp.int32),
            pltpu.VMEM((2, _W, 2 * D), jnp.float32),
            pltpu.SemaphoreType.DMA,
            pltpu.SemaphoreType.DMA,
        ],
    )
    def gather_kernel(table_hbm, idx_hbm, out_hbm, idx_v, rows_v, sem0, sem1):
        wid = lax.axis_index("s") * _NC + lax.axis_index("c")
        base = wid * n_per_w
        pltpu.sync_copy(idx_hbm.at[pl.ds(base, n_per_w)], idx_v)

        def fire(c, buf, sem):
            win = idx_v.at[pl.ds(c * _W, _W)]
            pltpu.async_copy(table_hbm.at[win], rows_v.at[buf], sem)

        def drain(c, buf, sem):
            win = idx_v.at[pl.ds(c * _W, _W)]
            pltpu.make_async_copy(table_hbm.at[win], rows_v.at[buf], sem).wait()
            pltpu.sync_copy(rows_v.at[buf], out_hbm.at[pl.ds(base + c * _W, _W)])

        fire(0, 0, sem0)

        @pl.loop(0, chunks // 2 - 1)
        def _(h):
            c = 2 * h
            fire(c + 1, 1, sem1)
            drain(c, 0, sem0)
            fire(c + 2, 0, sem0)
            drain(c + 1, 1, sem1)

        fire(chunks - 1, 1, sem1)
        drain(chunks - 2, 0, sem0)
        drain(chunks - 1, 1, sem1)

    fused = gather_kernel(table_p, idx)
    out = fused.reshape(S, B, 2 * D)[:, :, :D]
    return jnp.transpose(out, (1, 0, 2))
